# baseline (speedup 1.0000x reference)
import functools

import jax
import jax.numpy as jnp
from jax import lax
from jax.experimental import pallas as pl
from jax.experimental.pallas import tpu as pltpu
from jax.experimental.pallas import tpu_sc as plsc

B = 4096
D = 64
NSMALL = 4
NSONG = 1000000
NBUF = 6


_TBLK = 2048


def _pad_t_body(i0, i1, i2, o0, o1, o2):
    for i_ref, o_ref in ((i0, o0), (i1, o1), (i2, o2)):
        o_ref[:, :D] = i_ref[...].T


def _make_pad_t(n):
    grid = (n + _TBLK - 1) // _TBLK
    ispec = pl.BlockSpec((D, _TBLK), lambda i: (0, i))
    ospec = pl.BlockSpec((_TBLK, 2 * D), lambda i: (i, 0))
    oshape = jax.ShapeDtypeStruct((n, 2 * D), jnp.float32)
    return pl.pallas_call(
        _pad_t_body,
        grid=(grid,),
        in_specs=[ispec, ispec, ispec],
        out_specs=[ospec, ospec, ospec],
        out_shape=[oshape, oshape, oshape],
        compiler_params=pltpu.CompilerParams(
            dimension_semantics=("arbitrary",),
        ),
    )


def _make_sc_small_gather():
    info = plsc.get_sparse_core_info()
    nw = info.num_cores * info.num_subcores
    bpw = B // nw

    mesh = plsc.VectorSubcoreMesh(core_axis_name="c", subcore_axis_name="s")

    @functools.partial(
        pl.kernel,
        mesh=mesh,
        out_type=jax.ShapeDtypeStruct((NSMALL, B, 2 * D), jnp.float32),
        scratch_types=[
            pltpu.VMEM((NSMALL, bpw), jnp.int32),
            pltpu.VMEM((NSMALL, bpw, 2 * D), jnp.float32),
            pltpu.SemaphoreType.DMA,
            pltpu.SemaphoreType.DMA,
            pltpu.SemaphoreType.DMA,
        ],
        compiler_params=pltpu.CompilerParams(skip_device_barrier=True),
    )
    def gather_kernel(t0, t1, t2, t3, i0, i1, i2, i3,
                      out, idx_v, rows_v, semi, semg, semw):
        wid = lax.axis_index("s") * info.num_cores + lax.axis_index("c")
        base = wid * bpw
        tbls = (t0, t1, t2, t3)
        for t, ids in enumerate((i0, i1, i2, i3)):
            pltpu.make_async_copy(
                ids.at[pl.ds(base, bpw)], idx_v.at[t], semi).start()
        for t in range(NSMALL):
            pltpu.make_async_copy(
                i0.at[pl.ds(base, bpw)], idx_v.at[t], semi).wait()
            pltpu.make_async_copy(
                tbls[t].at[idx_v.at[t]], rows_v.at[t], semg).start()
        for t in range(NSMALL):
            pltpu.make_async_copy(
                tbls[t].at[pl.ds(0, bpw)], rows_v.at[t], semg).wait()
            pltpu.make_async_copy(
                rows_v.at[t], out.at[t, pl.ds(base, bpw), :], semw).start()
        for t in range(NSMALL):
            pltpu.make_async_copy(
                rows_v.at[t], out.at[t, pl.ds(base, bpw), :], semw).wait()

    return gather_kernel


def _make_sc_song_gather():
    info = plsc.get_sparse_core_info()
    nw = info.num_cores * info.num_subcores
    bpw = B // nw

    mesh = plsc.VectorSubcoreMesh(core_axis_name="c", subcore_axis_name="s")

    @functools.partial(
        pl.kernel,
        mesh=mesh,
        out_type=jax.ShapeDtypeStruct((B * D,), jnp.float32),
        scratch_types=[
            pltpu.VMEM((bpw + 32,), jnp.int32),
            pltpu.VMEM((12, 8, 8, 128), jnp.float32),
            pltpu.VMEM((bpw * D,), jnp.float32),
            pltpu.SemaphoreType.DMA,
            pltpu.SemaphoreType.DMA,
        ],
        compiler_params=pltpu.CompilerParams(needs_layout_passes=False,
                                             skip_device_barrier=True),
    )
    def song_kernel(v3, ids, out, idx_v, bufs, out_v, sem_a, sem_b):
        wid = lax.axis_index("s") * info.num_cores + lax.axis_index("c")
        base = wid * bpw
        pltpu.sync_copy(ids.at[pl.ds(base, bpw)], idx_v.at[pl.ds(0, bpw)])
        f16 = lax.iota(jnp.int32, 16)
        NB, AHEAD = 12, 10

        def idx_at(j):
            vec = idx_v[pl.ds(j, 16)]
            return jnp.squeeze(lax.slice(vec, (0,), (1,)))

        def fire(j, b, sem):
            col = pl.multiple_of((idx_at(j) >> 7) * 128, 128)
            pltpu.make_async_copy(
                v3.at[:, :, pl.ds(col, 128)], bufs.at[b], sem).start()

        for j in range(AHEAD):
            fire(j, j, sem_a if j % 2 == 0 else sem_b)

        def body2(j, sem):
            pltpu.make_async_copy(
                v3.at[:, :, pl.ds(0, 128)], bufs.at[0], sem).wait()
            r = idx_at(j)
            lane_vec = jnp.full((16,), r & 127, jnp.int32)
            b_vec = jnp.full((16,), j % NB, jnp.int32)
            for g in range(4):
                f = f16 + (16 * g)
                vals = plsc.load_gather(
                    bufs, [b_vec, f >> 3, f & 7, lane_vec])
                out_v[pl.ds(j * D + 16 * g, 16)] = vals

            @pl.when(j + AHEAD < bpw)
            def _():
                fire(j + AHEAD, (j + AHEAD) % NB, sem)

        def body(p, _):
            body2(2 * p, sem_a)
            body2(2 * p + 1, sem_b)
            return 0

        lax.fori_loop(0, bpw // 2, body, 0)
        pltpu.sync_copy(out_v, out.at[pl.ds(base * D, bpw * D)])

    return song_kernel


_sc_cached = {}


def _get_sc(name):
    if name not in _sc_cached:
        _sc_cached[name] = (_make_sc_small_gather() if name == "small"
                            else _make_sc_song_gather())
    return _sc_cached[name]


_BM = 512


def _mlp_body(x_ref, se_ref, pc_ref, wf_ref, wt1p_ref, wts_ref, w1pc_ref,
              b1_ref, wt2_ref, b2_ref, wt3_ref, b3_ref, v3_ref, ww_ref,
              c_ref, out_ref):
    f32 = jnp.float32
    a1 = pc_ref[...] * w1pc_ref[...] + b1_ref[...]
    for t in range(NSMALL):
        a1 = a1 + jnp.dot(x_ref[t][:, :D], wt1p_ref[t],
                          preferred_element_type=f32)
    a1 = a1 + jnp.dot(se_ref[...], wts_ref[...], preferred_element_type=f32)
    h1 = jnp.maximum(a1, 0.0)
    a2 = jnp.dot(h1, wt2_ref[...], preferred_element_type=f32) + b2_ref[...]
    h2 = jnp.maximum(a2, 0.0)
    a3 = jnp.dot(h2, wt3_ref[...], preferred_element_type=f32) + b3_ref[...]
    h3 = jnp.maximum(a3, 0.0)
    out = jnp.dot(h3, v3_ref[...], preferred_element_type=f32)
    out = out + jnp.dot(wf_ref[...], ww_ref[...], preferred_element_type=f32)
    out_ref[...] = out + c_ref[...]


def _full(shape):
    nd = len(shape)
    return pl.BlockSpec(shape, lambda i: (0,) * nd)


def _make_mlp_call(interpret=False):
    return pl.pallas_call(
        _mlp_body,
        grid=(B // _BM,),
        in_specs=[
            pl.BlockSpec((NSMALL, _BM, 2 * D), lambda i: (0, i, 0)),
            pl.BlockSpec((_BM, D), lambda i: (i, 0)),
            pl.BlockSpec((_BM, 1), lambda i: (i, 0)),
            pl.BlockSpec((_BM, 5), lambda i: (i, 0)),
            _full((NSMALL, D, 256)),
            _full((D, 256)),
            _full((1, 256)),
            _full((1, 256)),
            _full((256, 128)),
            _full((1, 128)),
            _full((128, 64)),
            _full((1, 64)),
            _full((64, 1)),
            _full((5, 1)),
            _full((1, 1)),
        ],
        out_specs=pl.BlockSpec((_BM, 1), lambda i: (i, 0)),
        out_shape=jax.ShapeDtypeStruct((B, 1), jnp.float32),
        compiler_params=pltpu.CompilerParams(
            dimension_semantics=("arbitrary",),
        ),
        interpret=interpret,
    )


_mlp_call = _make_mlp_call()


def kernel(wide_features, user_ids, song_ids, artist_ids, release_ids,
           year_ids, play_count, user_table, song_table, artist_table,
           release_table, year_table, W_wide, b_wide, W1, b1, g1, be1, mu1,
           var1, W2, b2, g2, be2, mu2, var2, W3, b3, g3, be3, mu3, var3,
           W_final, b_final):
    f32 = jnp.float32
    eps = 1e-5

    song_v3 = song_table.T.reshape(8, 8, NSONG)
    song_flat = _get_sc("song")(song_v3, song_ids.astype(jnp.int32))
    se = song_flat.reshape(B, D)

    up, ap, rp = _make_pad_t(100000)(
        user_table.T, artist_table.T, release_table.T)
    yp = jnp.pad(year_table.astype(f32), ((0, 0), (0, D)))
    ids4 = [x.astype(jnp.int32) for x in
            (user_ids, artist_ids, release_ids, year_ids)]
    x4 = _get_sc("small")(up, ap, rp, yp, *ids4)

    s1 = g1 / jnp.sqrt(var1 + eps)
    sh1 = be1 - mu1 * s1
    s2 = g2 / jnp.sqrt(var2 + eps)
    sh2 = be2 - mu2 * s2
    s3 = g3 / jnp.sqrt(var3 + eps)
    sh3 = be3 - mu3 * s3

    w1t = W1.T.astype(f32)
    wt1p = jnp.stack([w1t[0 * D:1 * D], w1t[2 * D:3 * D],
                      w1t[3 * D:4 * D], w1t[4 * D:5 * D]])
    wts = w1t[1 * D:2 * D]
    w1pc = w1t[5 * D:]
    b1r = b1[None, :].astype(f32)

    wt2 = (W2 * s1[None, :]).T.astype(f32)
    b2f = (b2 + W2 @ sh1)[None, :].astype(f32)
    wt3 = (W3 * s2[None, :]).T.astype(f32)
    b3f = (b3 + W3 @ sh2)[None, :].astype(f32)

    wf_emb = W_final[0, :D]
    v3 = (s3 * wf_emb)[:, None].astype(f32)
    ww = (W_wide[0] * W_final[0, D])[:, None].astype(f32)
    c = (b_final[0] + sh3 @ wf_emb + b_wide[0] * W_final[0, D])
    c = jnp.reshape(c, (1, 1)).astype(f32)

    pc = play_count[:, None].astype(f32)

    return _mlp_call(x4, se, pc, wide_features.astype(f32), wt1p, wts, w1pc,
                     b1r, wt2, b2f, wt3, b3f, v3, ww, c)


# speedup vs baseline: 1.0011x; 1.0011x over previous
import functools

import jax
import jax.numpy as jnp
from jax import lax
from jax.experimental import pallas as pl
from jax.experimental.pallas import tpu as pltpu
from jax.experimental.pallas import tpu_sc as plsc

B = 4096
D = 64
NSMALL = 4
NSONG = 1000000
NBUF = 6


_TBLK = 2048


def _pad_t_body(i0, i1, i2, o0, o1, o2):
    for i_ref, o_ref in ((i0, o0), (i1, o1), (i2, o2)):
        o_ref[:, :D] = i_ref[...].T


def _make_pad_t(n):
    grid = (n + _TBLK - 1) // _TBLK
    ispec = pl.BlockSpec((D, _TBLK), lambda i: (0, i))
    ospec = pl.BlockSpec((_TBLK, 2 * D), lambda i: (i, 0))
    oshape = jax.ShapeDtypeStruct((n, 2 * D), jnp.float32)
    return pl.pallas_call(
        _pad_t_body,
        grid=(grid,),
        in_specs=[ispec, ispec, ispec],
        out_specs=[ospec, ospec, ospec],
        out_shape=[oshape, oshape, oshape],
        compiler_params=pltpu.CompilerParams(
            dimension_semantics=("arbitrary",),
        ),
    )


def _make_sc_small_gather():
    info = plsc.get_sparse_core_info()
    nw = info.num_cores * info.num_subcores
    bpw = B // nw

    mesh = plsc.VectorSubcoreMesh(core_axis_name="c", subcore_axis_name="s")

    @functools.partial(
        pl.kernel,
        mesh=mesh,
        out_type=jax.ShapeDtypeStruct((NSMALL, B, 2 * D), jnp.float32),
        scratch_types=[
            pltpu.VMEM((NSMALL, bpw), jnp.int32),
            pltpu.VMEM((NSMALL, bpw, 2 * D), jnp.float32),
            pltpu.SemaphoreType.DMA,
            pltpu.SemaphoreType.DMA,
            pltpu.SemaphoreType.DMA,
        ],
        compiler_params=pltpu.CompilerParams(skip_device_barrier=True),
    )
    def gather_kernel(t0, t1, t2, t3, i0, i1, i2, i3,
                      out, idx_v, rows_v, semi, semg, semw):
        wid = lax.axis_index("s") * info.num_cores + lax.axis_index("c")
        base = wid * bpw
        tbls = (t0, t1, t2, t3)
        for t, ids in enumerate((i0, i1, i2, i3)):
            pltpu.make_async_copy(
                ids.at[pl.ds(base, bpw)], idx_v.at[t], semi).start()
        for t in range(NSMALL):
            pltpu.make_async_copy(
                i0.at[pl.ds(base, bpw)], idx_v.at[t], semi).wait()
            pltpu.make_async_copy(
                tbls[t].at[idx_v.at[t]], rows_v.at[t], semg).start()
        for t in range(NSMALL):
            pltpu.make_async_copy(
                tbls[t].at[pl.ds(0, bpw)], rows_v.at[t], semg).wait()
            pltpu.make_async_copy(
                rows_v.at[t], out.at[t, pl.ds(base, bpw), :], semw).start()
        for t in range(NSMALL):
            pltpu.make_async_copy(
                rows_v.at[t], out.at[t, pl.ds(base, bpw), :], semw).wait()

    return gather_kernel


def _make_sc_song_gather():
    info = plsc.get_sparse_core_info()
    nw = info.num_cores * info.num_subcores
    bpw = B // nw

    mesh = plsc.VectorSubcoreMesh(core_axis_name="c", subcore_axis_name="s")

    @functools.partial(
        pl.kernel,
        mesh=mesh,
        out_type=jax.ShapeDtypeStruct((B * D,), jnp.float32),
        scratch_types=[
            pltpu.VMEM((bpw + 32,), jnp.int32),
            pltpu.VMEM((12, 8, 8, 128), jnp.float32),
            pltpu.VMEM((bpw * D,), jnp.float32),
            pltpu.SemaphoreType.DMA,
        ],
        compiler_params=pltpu.CompilerParams(needs_layout_passes=False,
                                             skip_device_barrier=True),
    )
    def song_kernel(v3, ids, out, idx_v, bufs, out_v, sem):
        wid = lax.axis_index("s") * info.num_cores + lax.axis_index("c")
        base = wid * bpw
        pltpu.sync_copy(ids.at[pl.ds(base, bpw)], idx_v.at[pl.ds(0, bpw)])
        f16 = lax.iota(jnp.int32, 16)
        NB, AHEAD = 12, 11

        def idx_at(j):
            vec = idx_v[pl.ds(j, 16)]
            return jnp.squeeze(lax.slice(vec, (0,), (1,)))

        def fire(j, b, sem):
            col = pl.multiple_of((idx_at(j) >> 7) * 128, 128)
            pltpu.make_async_copy(
                v3.at[:, :, pl.ds(col, 128)], bufs.at[b], sem).start()

        for j in range(AHEAD):
            fire(j, j, sem)

        def body(j, _):
            pltpu.make_async_copy(
                v3.at[:, :, pl.ds(0, 128)], bufs.at[0], sem).wait()
            r = idx_at(j)
            lane_vec = jnp.full((16,), r & 127, jnp.int32)
            b_vec = jnp.full((16,), j % NB, jnp.int32)
            for g in range(4):
                f = f16 + (16 * g)
                vals = plsc.load_gather(
                    bufs, [b_vec, f >> 3, f & 7, lane_vec])
                out_v[pl.ds(j * D + 16 * g, 16)] = vals

            @pl.when(j + AHEAD < bpw)
            def _():
                fire(j + AHEAD, (j + AHEAD) % NB, sem)

            return 0

        lax.fori_loop(0, bpw, body, 0)
        pltpu.sync_copy(out_v, out.at[pl.ds(base * D, bpw * D)])

    return song_kernel


_sc_cached = {}


def _get_sc(name):
    if name not in _sc_cached:
        _sc_cached[name] = (_make_sc_small_gather() if name == "small"
                            else _make_sc_song_gather())
    return _sc_cached[name]


_BM = 512


def _mlp_body(x_ref, se_ref, pc_ref, wf_ref, wt1p_ref, wts_ref, w1pc_ref,
              b1_ref, wt2_ref, b2_ref, wt3_ref, b3_ref, v3_ref, ww_ref,
              c_ref, out_ref):
    f32 = jnp.float32
    a1 = pc_ref[...] * w1pc_ref[...] + b1_ref[...]
    for t in range(NSMALL):
        a1 = a1 + jnp.dot(x_ref[t][:, :D], wt1p_ref[t],
                          preferred_element_type=f32)
    a1 = a1 + jnp.dot(se_ref[...], wts_ref[...], preferred_element_type=f32)
    h1 = jnp.maximum(a1, 0.0)
    a2 = jnp.dot(h1, wt2_ref[...], preferred_element_type=f32) + b2_ref[...]
    h2 = jnp.maximum(a2, 0.0)
    a3 = jnp.dot(h2, wt3_ref[...], preferred_element_type=f32) + b3_ref[...]
    h3 = jnp.maximum(a3, 0.0)
    out = jnp.dot(h3, v3_ref[...], preferred_element_type=f32)
    out = out + jnp.dot(wf_ref[...], ww_ref[...], preferred_element_type=f32)
    out_ref[...] = out + c_ref[...]


def _full(shape):
    nd = len(shape)
    return pl.BlockSpec(shape, lambda i: (0,) * nd)


def _make_mlp_call(interpret=False):
    return pl.pallas_call(
        _mlp_body,
        grid=(B // _BM,),
        in_specs=[
            pl.BlockSpec((NSMALL, _BM, 2 * D), lambda i: (0, i, 0)),
            pl.BlockSpec((_BM, D), lambda i: (i, 0)),
            pl.BlockSpec((_BM, 1), lambda i: (i, 0)),
            pl.BlockSpec((_BM, 5), lambda i: (i, 0)),
            _full((NSMALL, D, 256)),
            _full((D, 256)),
            _full((1, 256)),
            _full((1, 256)),
            _full((256, 128)),
            _full((1, 128)),
            _full((128, 64)),
            _full((1, 64)),
            _full((64, 1)),
            _full((5, 1)),
            _full((1, 1)),
        ],
        out_specs=pl.BlockSpec((_BM, 1), lambda i: (i, 0)),
        out_shape=jax.ShapeDtypeStruct((B, 1), jnp.float32),
        compiler_params=pltpu.CompilerParams(
            dimension_semantics=("arbitrary",),
        ),
        interpret=interpret,
    )


_mlp_call = _make_mlp_call()


def kernel(wide_features, user_ids, song_ids, artist_ids, release_ids,
           year_ids, play_count, user_table, song_table, artist_table,
           release_table, year_table, W_wide, b_wide, W1, b1, g1, be1, mu1,
           var1, W2, b2, g2, be2, mu2, var2, W3, b3, g3, be3, mu3, var3,
           W_final, b_final):
    f32 = jnp.float32
    eps = 1e-5

    song_v3 = song_table.T.reshape(8, 8, NSONG)
    song_flat = _get_sc("song")(song_v3, song_ids.astype(jnp.int32))
    se = song_flat.reshape(B, D)

    up, ap, rp = _make_pad_t(100000)(
        user_table.T, artist_table.T, release_table.T)
    yp = jnp.pad(year_table.astype(f32), ((0, 0), (0, D)))
    ids4 = [x.astype(jnp.int32) for x in
            (user_ids, artist_ids, release_ids, year_ids)]
    x4 = _get_sc("small")(up, ap, rp, yp, *ids4)

    s1 = g1 / jnp.sqrt(var1 + eps)
    sh1 = be1 - mu1 * s1
    s2 = g2 / jnp.sqrt(var2 + eps)
    sh2 = be2 - mu2 * s2
    s3 = g3 / jnp.sqrt(var3 + eps)
    sh3 = be3 - mu3 * s3

    w1t = W1.T.astype(f32)
    wt1p = jnp.stack([w1t[0 * D:1 * D], w1t[2 * D:3 * D],
                      w1t[3 * D:4 * D], w1t[4 * D:5 * D]])
    wts = w1t[1 * D:2 * D]
    w1pc = w1t[5 * D:]
    b1r = b1[None, :].astype(f32)

    wt2 = (W2 * s1[None, :]).T.astype(f32)
    b2f = (b2 + W2 @ sh1)[None, :].astype(f32)
    wt3 = (W3 * s2[None, :]).T.astype(f32)
    b3f = (b3 + W3 @ sh2)[None, :].astype(f32)

    wf_emb = W_final[0, :D]
    v3 = (s3 * wf_emb)[:, None].astype(f32)
    ww = (W_wide[0] * W_final[0, D])[:, None].astype(f32)
    c = (b_final[0] + sh3 @ wf_emb + b_wide[0] * W_final[0, D])
    c = jnp.reshape(c, (1, 1)).astype(f32)

    pc = play_count[:, None].astype(f32)

    return _mlp_call(x4, se, pc, wide_features.astype(f32), wt1p, wts, w1pc,
                     b1r, wt2, b2f, wt3, b3f, v3, ww, c)


# speedup vs baseline: 1.0075x; 1.0065x over previous
import functools

import jax
import jax.numpy as jnp
from jax import lax
from jax.experimental import pallas as pl
from jax.experimental.pallas import tpu as pltpu
from jax.experimental.pallas import tpu_sc as plsc

B = 4096
D = 64
NSMALL = 4
NSONG = 1000000
NBUF = 6


_TBLK = 2048


def _pad_t_body(i0, i1, i2, o0, o1, o2):
    for i_ref, o_ref in ((i0, o0), (i1, o1), (i2, o2)):
        o_ref[:, :D] = i_ref[...].T


def _make_pad_t(n):
    grid = (n + _TBLK - 1) // _TBLK
    ispec = pl.BlockSpec((D, _TBLK), lambda i: (0, i))
    ospec = pl.BlockSpec((_TBLK, 2 * D), lambda i: (i, 0))
    oshape = jax.ShapeDtypeStruct((n, 2 * D), jnp.float32)
    return pl.pallas_call(
        _pad_t_body,
        grid=(grid,),
        in_specs=[ispec, ispec, ispec],
        out_specs=[ospec, ospec, ospec],
        out_shape=[oshape, oshape, oshape],
        compiler_params=pltpu.CompilerParams(
            dimension_semantics=("arbitrary",),
        ),
    )


def _make_sc_small_gather():
    info = plsc.get_sparse_core_info()
    nw = info.num_cores * info.num_subcores
    bpw = B // nw

    mesh = plsc.VectorSubcoreMesh(core_axis_name="c", subcore_axis_name="s")

    @functools.partial(
        pl.kernel,
        mesh=mesh,
        out_type=jax.ShapeDtypeStruct((NSMALL, B, 2 * D), jnp.float32),
        scratch_types=[
            pltpu.VMEM((bpw,), jnp.int32),
            pltpu.VMEM((bpw, 2 * D), jnp.float32),
            pltpu.SemaphoreType.DMA,
        ],
        compiler_params=pltpu.CompilerParams(skip_device_barrier=True),
    )
    def gather_kernel(t0, t1, t2, t3, i0, i1, i2, i3,
                      out, idx_v, rows_v, sem):
        wid = lax.axis_index("s") * info.num_cores + lax.axis_index("c")
        base = wid * bpw
        for t, (tbl, ids) in enumerate(
                ((t0, i0), (t1, i1), (t2, i2), (t3, i3))):
            pltpu.sync_copy(ids.at[pl.ds(base, bpw)], idx_v)
            pltpu.async_copy(tbl.at[idx_v], rows_v, sem).wait()
            pltpu.sync_copy(rows_v, out.at[t, pl.ds(base, bpw), :])

    return gather_kernel


def _make_sc_song_gather():
    info = plsc.get_sparse_core_info()
    nw = info.num_cores * info.num_subcores
    bpw = B // nw

    mesh = plsc.VectorSubcoreMesh(core_axis_name="c", subcore_axis_name="s")

    @functools.partial(
        pl.kernel,
        mesh=mesh,
        out_type=jax.ShapeDtypeStruct((B * D,), jnp.float32),
        scratch_types=[
            pltpu.VMEM((bpw + 32,), jnp.int32),
            pltpu.VMEM((12, 8, 8, 128), jnp.float32),
            pltpu.VMEM((bpw * D,), jnp.float32),
            pltpu.SemaphoreType.DMA,
        ],
        compiler_params=pltpu.CompilerParams(needs_layout_passes=False,
                                             skip_device_barrier=True),
    )
    def song_kernel(v3, ids, out, idx_v, bufs, out_v, sem):
        wid = lax.axis_index("s") * info.num_cores + lax.axis_index("c")
        base = wid * bpw
        pltpu.sync_copy(ids.at[pl.ds(base, bpw)], idx_v.at[pl.ds(0, bpw)])
        f16 = lax.iota(jnp.int32, 16)
        NB, AHEAD = 12, 11

        def idx_at(j):
            vec = idx_v[pl.ds(j, 16)]
            return jnp.squeeze(lax.slice(vec, (0,), (1,)))

        def fire(j, b, sem):
            col = pl.multiple_of((idx_at(j) >> 7) * 128, 128)
            pltpu.make_async_copy(
                v3.at[:, :, pl.ds(col, 128)], bufs.at[b], sem).start()

        for j in range(AHEAD):
            fire(j, j, sem)

        def body(j, _):
            pltpu.make_async_copy(
                v3.at[:, :, pl.ds(0, 128)], bufs.at[0], sem).wait()
            r = idx_at(j)
            lane_vec = jnp.full((16,), r & 127, jnp.int32)
            b_vec = jnp.full((16,), j % NB, jnp.int32)
            for g in range(4):
                f = f16 + (16 * g)
                vals = plsc.load_gather(
                    bufs, [b_vec, f >> 3, f & 7, lane_vec])
                out_v[pl.ds(j * D + 16 * g, 16)] = vals

            @pl.when(j + AHEAD < bpw)
            def _():
                fire(j + AHEAD, (j + AHEAD) % NB, sem)

            return 0

        lax.fori_loop(0, bpw, body, 0)
        pltpu.sync_copy(out_v, out.at[pl.ds(base * D, bpw * D)])

    return song_kernel


_sc_cached = {}


def _get_sc(name):
    if name not in _sc_cached:
        _sc_cached[name] = (_make_sc_small_gather() if name == "small"
                            else _make_sc_song_gather())
    return _sc_cached[name]


_BM = 512


def _mlp_body(x_ref, se_ref, pc_ref, wf_ref, wt1p_ref, wts_ref, w1pc_ref,
              b1_ref, wt2_ref, b2_ref, wt3_ref, b3_ref, v3_ref, ww_ref,
              c_ref, out_ref):
    f32 = jnp.float32
    a1 = pc_ref[...] * w1pc_ref[...] + b1_ref[...]
    for t in range(NSMALL):
        a1 = a1 + jnp.dot(x_ref[t][:, :D], wt1p_ref[t],
                          preferred_element_type=f32)
    a1 = a1 + jnp.dot(se_ref[...], wts_ref[...], preferred_element_type=f32)
    h1 = jnp.maximum(a1, 0.0)
    a2 = jnp.dot(h1, wt2_ref[...], preferred_element_type=f32) + b2_ref[...]
    h2 = jnp.maximum(a2, 0.0)
    a3 = jnp.dot(h2, wt3_ref[...], preferred_element_type=f32) + b3_ref[...]
    h3 = jnp.maximum(a3, 0.0)
    out = jnp.dot(h3, v3_ref[...], preferred_element_type=f32)
    out = out + jnp.dot(wf_ref[...], ww_ref[...], preferred_element_type=f32)
    out_ref[...] = out + c_ref[...]


def _full(shape):
    nd = len(shape)
    return pl.BlockSpec(shape, lambda i: (0,) * nd)


def _make_mlp_call(interpret=False):
    return pl.pallas_call(
        _mlp_body,
        grid=(B // _BM,),
        in_specs=[
            pl.BlockSpec((NSMALL, _BM, 2 * D), lambda i: (0, i, 0)),
            pl.BlockSpec((_BM, D), lambda i: (i, 0)),
            pl.BlockSpec((_BM, 1), lambda i: (i, 0)),
            pl.BlockSpec((_BM, 5), lambda i: (i, 0)),
            _full((NSMALL, D, 256)),
            _full((D, 256)),
            _full((1, 256)),
            _full((1, 256)),
            _full((256, 128)),
            _full((1, 128)),
            _full((128, 64)),
            _full((1, 64)),
            _full((64, 1)),
            _full((5, 1)),
            _full((1, 1)),
        ],
        out_specs=pl.BlockSpec((_BM, 1), lambda i: (i, 0)),
        out_shape=jax.ShapeDtypeStruct((B, 1), jnp.float32),
        compiler_params=pltpu.CompilerParams(
            dimension_semantics=("arbitrary",),
        ),
        interpret=interpret,
    )


_mlp_call = _make_mlp_call()


def kernel(wide_features, user_ids, song_ids, artist_ids, release_ids,
           year_ids, play_count, user_table, song_table, artist_table,
           release_table, year_table, W_wide, b_wide, W1, b1, g1, be1, mu1,
           var1, W2, b2, g2, be2, mu2, var2, W3, b3, g3, be3, mu3, var3,
           W_final, b_final):
    f32 = jnp.float32
    eps = 1e-5

    song_v3 = song_table.T.reshape(8, 8, NSONG)
    song_flat = _get_sc("song")(song_v3, song_ids.astype(jnp.int32))
    se = song_flat.reshape(B, D)

    up, ap, rp = _make_pad_t(100000)(
        user_table.T, artist_table.T, release_table.T)
    yp = jnp.pad(year_table.astype(f32), ((0, 0), (0, D)))
    ids4 = [x.astype(jnp.int32) for x in
            (user_ids, artist_ids, release_ids, year_ids)]
    x4 = _get_sc("small")(up, ap, rp, yp, *ids4)

    s1 = g1 / jnp.sqrt(var1 + eps)
    sh1 = be1 - mu1 * s1
    s2 = g2 / jnp.sqrt(var2 + eps)
    sh2 = be2 - mu2 * s2
    s3 = g3 / jnp.sqrt(var3 + eps)
    sh3 = be3 - mu3 * s3

    w1t = W1.T.astype(f32)
    wt1p = jnp.stack([w1t[0 * D:1 * D], w1t[2 * D:3 * D],
                      w1t[3 * D:4 * D], w1t[4 * D:5 * D]])
    wts = w1t[1 * D:2 * D]
    w1pc = w1t[5 * D:]
    b1r = b1[None, :].astype(f32)

    wt2 = (W2 * s1[None, :]).T.astype(f32)
    b2f = (b2 + W2 @ sh1)[None, :].astype(f32)
    wt3 = (W3 * s2[None, :]).T.astype(f32)
    b3f = (b3 + W3 @ sh2)[None, :].astype(f32)

    wf_emb = W_final[0, :D]
    v3 = (s3 * wf_emb)[:, None].astype(f32)
    ww = (W_wide[0] * W_final[0, D])[:, None].astype(f32)
    c = (b_final[0] + sh3 @ wf_emb + b_wide[0] * W_final[0, D])
    c = jnp.reshape(c, (1, 1)).astype(f32)

    pc = play_count[:, None].astype(f32)

    return _mlp_call(x4, se, pc, wide_features.astype(f32), wt1p, wts, w1pc,
                     b1r, wt2, b2f, wt3, b3f, v3, ww, c)


# speedup vs baseline: 1.0088x; 1.0013x over previous
import functools

import jax
import jax.numpy as jnp
from jax import lax
from jax.experimental import pallas as pl
from jax.experimental.pallas import tpu as pltpu
from jax.experimental.pallas import tpu_sc as plsc

B = 4096
D = 64
NSMALL = 4
NSONG = 1000000


_TBLK = 2048


def _pad_t_body(i0, i1, i2, o0, o1, o2):
    for i_ref, o_ref in ((i0, o0), (i1, o1), (i2, o2)):
        o_ref[:, :D] = i_ref[...].T


def _make_pad_t(n):
    grid = (n + _TBLK - 1) // _TBLK
    ispec = pl.BlockSpec((D, _TBLK), lambda i: (0, i))
    ospec = pl.BlockSpec((_TBLK, 2 * D), lambda i: (i, 0))
    oshape = jax.ShapeDtypeStruct((n, 2 * D), jnp.float32)
    return pl.pallas_call(
        _pad_t_body,
        grid=(grid,),
        in_specs=[ispec, ispec, ispec],
        out_specs=[ospec, ospec, ospec],
        out_shape=[oshape, oshape, oshape],
        compiler_params=pltpu.CompilerParams(
            dimension_semantics=("arbitrary",),
        ),
    )


def _make_sc_small_gather():
    info = plsc.get_sparse_core_info()
    nw = info.num_cores * info.num_subcores
    bpw = B // nw

    mesh = plsc.VectorSubcoreMesh(core_axis_name="c", subcore_axis_name="s")

    @functools.partial(
        pl.kernel,
        mesh=mesh,
        out_type=jax.ShapeDtypeStruct((NSMALL, B, 2 * D), jnp.float32),
        scratch_types=[
            pltpu.VMEM((bpw,), jnp.int32),
            pltpu.VMEM((bpw, 2 * D), jnp.float32),
            pltpu.SemaphoreType.DMA,
        ],
        compiler_params=pltpu.CompilerParams(skip_device_barrier=True),
    )
    def gather_kernel(t0, t1, t2, t3, i0, i1, i2, i3,
                      out, idx_v, rows_v, sem):
        wid = lax.axis_index("s") * info.num_cores + lax.axis_index("c")
        base = wid * bpw
        for t, (tbl, ids) in enumerate(
                ((t0, i0), (t1, i1), (t2, i2), (t3, i3))):
            pltpu.sync_copy(ids.at[pl.ds(base, bpw)], idx_v)
            pltpu.async_copy(tbl.at[idx_v], rows_v, sem).wait()
            pltpu.sync_copy(rows_v, out.at[t, pl.ds(base, bpw), :])

    return gather_kernel


def _make_sc_song_gather():
    info = plsc.get_sparse_core_info()
    nw = info.num_cores * info.num_subcores
    bpw = B // nw

    mesh = plsc.VectorSubcoreMesh(core_axis_name="c", subcore_axis_name="s")

    @functools.partial(
        pl.kernel,
        mesh=mesh,
        out_type=jax.ShapeDtypeStruct((B * D,), jnp.float32),
        scratch_types=[
            pltpu.VMEM((bpw + 32,), jnp.int32),
            pltpu.VMEM((12, 8, 8, 128), jnp.float32),
            pltpu.VMEM((bpw * D,), jnp.float32),
            pltpu.SemaphoreType.DMA,
        ],
        compiler_params=pltpu.CompilerParams(needs_layout_passes=False,
                                             skip_device_barrier=True),
    )
    def song_kernel(v3, ids, out, idx_v, bufs, out_v, sem):
        wid = lax.axis_index("s") * info.num_cores + lax.axis_index("c")
        base = wid * bpw
        pltpu.sync_copy(ids.at[pl.ds(base, bpw)], idx_v.at[pl.ds(0, bpw)])
        f16 = lax.iota(jnp.int32, 16)
        NB, AHEAD = 12, 11

        def idx_at(j):
            vec = idx_v[pl.ds(j, 16)]
            return jnp.squeeze(lax.slice(vec, (0,), (1,)))

        def fire(j, b, sem):
            col = pl.multiple_of((idx_at(j) >> 7) * 128, 128)
            pltpu.make_async_copy(
                v3.at[:, :, pl.ds(col, 128)], bufs.at[b], sem).start()

        for j in range(AHEAD):
            fire(j, j, sem)

        def body(j, _):
            pltpu.make_async_copy(
                v3.at[:, :, pl.ds(0, 128)], bufs.at[0], sem).wait()
            r = idx_at(j)
            lane_vec = jnp.full((16,), r & 127, jnp.int32)
            b_vec = jnp.full((16,), j % NB, jnp.int32)
            for g in range(4):
                f = f16 + (16 * g)
                vals = plsc.load_gather(
                    bufs, [b_vec, f >> 3, f & 7, lane_vec])
                out_v[pl.ds(j * D + 16 * g, 16)] = vals

            @pl.when(j + AHEAD < bpw)
            def _():
                fire(j + AHEAD, (j + AHEAD) % NB, sem)

            return 0

        lax.fori_loop(0, bpw, body, 0)
        pltpu.sync_copy(out_v, out.at[pl.ds(base * D, bpw * D)])

    return song_kernel


_sc_cached = {}


def _get_sc(name):
    if name not in _sc_cached:
        _sc_cached[name] = (_make_sc_small_gather() if name == "small"
                            else _make_sc_song_gather())
    return _sc_cached[name]


_BM = 512


def _mlp_body(x_ref, se_ref, pc_ref, wf_ref, wt1p_ref, wts_ref, w1pc_ref,
              b1_ref, wt2_ref, b2_ref, wt3_ref, b3_ref, v3_ref, ww_ref,
              c_ref, out_ref):
    f32 = jnp.float32
    a1 = pc_ref[...] * w1pc_ref[...] + b1_ref[...]
    for t in range(NSMALL):
        a1 = a1 + jnp.dot(x_ref[t][:, :D], wt1p_ref[t],
                          preferred_element_type=f32)
    a1 = a1 + jnp.dot(se_ref[...], wts_ref[...], preferred_element_type=f32)
    h1 = jnp.maximum(a1, 0.0)
    a2 = jnp.dot(h1, wt2_ref[...], preferred_element_type=f32) + b2_ref[...]
    h2 = jnp.maximum(a2, 0.0)
    a3 = jnp.dot(h2, wt3_ref[...], preferred_element_type=f32) + b3_ref[...]
    h3 = jnp.maximum(a3, 0.0)
    out = jnp.dot(h3, v3_ref[...], preferred_element_type=f32)
    out = out + jnp.dot(wf_ref[...], ww_ref[...], preferred_element_type=f32)
    out_ref[...] = out + c_ref[...]


def _full(shape):
    nd = len(shape)
    return pl.BlockSpec(shape, lambda i: (0,) * nd)


def _make_mlp_call(interpret=False):
    return pl.pallas_call(
        _mlp_body,
        grid=(B // _BM,),
        in_specs=[
            pl.BlockSpec((NSMALL, _BM, 2 * D), lambda i: (0, i, 0)),
            pl.BlockSpec((_BM, D), lambda i: (i, 0)),
            pl.BlockSpec((_BM, 1), lambda i: (i, 0)),
            pl.BlockSpec((_BM, 5), lambda i: (i, 0)),
            _full((NSMALL, D, 256)),
            _full((D, 256)),
            _full((1, 256)),
            _full((1, 256)),
            _full((256, 128)),
            _full((1, 128)),
            _full((128, 64)),
            _full((1, 64)),
            _full((64, 1)),
            _full((5, 1)),
            _full((1, 1)),
        ],
        out_specs=pl.BlockSpec((_BM, 1), lambda i: (i, 0)),
        out_shape=jax.ShapeDtypeStruct((B, 1), jnp.float32),
        compiler_params=pltpu.CompilerParams(
            dimension_semantics=("arbitrary",),
        ),
        interpret=interpret,
    )


_mlp_call = _make_mlp_call()


def kernel(wide_features, user_ids, song_ids, artist_ids, release_ids,
           year_ids, play_count, user_table, song_table, artist_table,
           release_table, year_table, W_wide, b_wide, W1, b1, g1, be1, mu1,
           var1, W2, b2, g2, be2, mu2, var2, W3, b3, g3, be3, mu3, var3,
           W_final, b_final):
    f32 = jnp.float32
    eps = 1e-5

    song_v3 = song_table.T.reshape(8, 8, NSONG)
    song_flat = _get_sc("song")(song_v3, song_ids.astype(jnp.int32))
    se = song_flat.reshape(B, D)

    up, ap, rp = _make_pad_t(100000)(
        user_table.T, artist_table.T, release_table.T)
    yp = jnp.pad(year_table.astype(f32), ((0, 0), (0, D)))
    ids4 = [x.astype(jnp.int32) for x in
            (user_ids, artist_ids, release_ids, year_ids)]
    x4 = _get_sc("small")(up, ap, rp, yp, *ids4)

    s1 = g1 / jnp.sqrt(var1 + eps)
    sh1 = be1 - mu1 * s1
    s2 = g2 / jnp.sqrt(var2 + eps)
    sh2 = be2 - mu2 * s2
    s3 = g3 / jnp.sqrt(var3 + eps)
    sh3 = be3 - mu3 * s3

    w1t = W1.T.astype(f32)
    wt1p = jnp.stack([w1t[0 * D:1 * D], w1t[2 * D:3 * D],
                      w1t[3 * D:4 * D], w1t[4 * D:5 * D]])
    wts = w1t[1 * D:2 * D]
    w1pc = w1t[5 * D:]
    b1r = b1[None, :].astype(f32)

    wt2 = (W2 * s1[None, :]).T.astype(f32)
    b2f = (b2 + W2 @ sh1)[None, :].astype(f32)
    wt3 = (W3 * s2[None, :]).T.astype(f32)
    b3f = (b3 + W3 @ sh2)[None, :].astype(f32)

    wf_emb = W_final[0, :D]
    v3 = (s3 * wf_emb)[:, None].astype(f32)
    ww = (W_wide[0] * W_final[0, D])[:, None].astype(f32)
    c = (b_final[0] + sh3 @ wf_emb + b_wide[0] * W_final[0, D])
    c = jnp.reshape(c, (1, 1)).astype(f32)

    pc = play_count[:, None].astype(f32)

    return _mlp_call(x4, se, pc, wide_features.astype(f32), wt1p, wts, w1pc,
                     b1r, wt2, b2f, wt3, b3f, v3, ww, c)
